# Initial kernel scaffold; baseline (speedup 1.0000x reference)
import jax
import jax.numpy as jnp
from jax.experimental import pallas as pl


def kernel(x, neuron_A, neuron_B, Wq, bq, Wk, bk, Wv, bv, Wp, bp):
    raise NotImplementedError("write your pallas kernel here")


# speedup vs baseline: 6.1343x; 6.1343x over previous
import functools
import math

import jax
import jax.numpy as jnp
from jax import lax
from jax.experimental import pallas as pl
from jax.experimental.pallas import tpu as pltpu
from jax.experimental.pallas import tpu_sc as plsc

N_NEURONS = 16384
D_MODEL = 1024
N_HEADS = 16
D_HEAD = 64
K = 16
RANK = 32
S = 2048

Q_BLK = 512
TOK_BLK = 128
NEU_BLK = 2048
N_NB = N_NEURONS // NEU_BLK
EXP_BLK = 4096
NEG = -3.0e38

_DN = (((1,), (1,)), ((), ()))


def _qkv_body(x_ref, wq_ref, wk_ref, wv_ref, bq_ref, bk_ref, bv_ref,
              q_ref, k_ref, v_ref):
    x = x_ref[...]
    q_ref[...] = (lax.dot_general(x, wq_ref[...], _DN,
                                  preferred_element_type=jnp.float32)
                  + bq_ref[...].reshape(1, D_HEAD))[None]
    k_ref[...] = (lax.dot_general(x, wk_ref[...], _DN,
                                  preferred_element_type=jnp.float32)
                  + bk_ref[...].reshape(1, D_HEAD))[None]
    v_ref[...] = (lax.dot_general(x, wv_ref[...], _DN,
                                  preferred_element_type=jnp.float32)
                  + bv_ref[...].reshape(1, D_HEAD))[None]


def _attn_body(q_ref, k_ref, v_ref, out_ref):
    q = q_ref[0]
    k = k_ref[0]
    v = v_ref[0]
    s = lax.dot_general(q, k, _DN, preferred_element_type=jnp.float32)
    s = s * jnp.float32(1.0 / math.sqrt(D_HEAD))
    C = 1024
    m_prev = jnp.full((Q_BLK, 1), -jnp.inf, jnp.float32)
    den_prev = jnp.zeros((Q_BLK, 1), jnp.float32)
    acc = jnp.zeros((Q_BLK, D_HEAD), jnp.float32)
    for j in range(S // C):
        sc = s[:, j * C:(j + 1) * C]
        mj = jnp.max(sc, axis=1, keepdims=True)
        m_new = jnp.maximum(m_prev, mj)
        delta = jnp.where(m_prev == m_new, jnp.float32(0.0), m_prev - m_new)
        corr = jnp.exp(delta)
        e = jnp.exp(sc - m_new)
        cd = corr * den_prev
        den_new = cd + jnp.sum(e, axis=1, keepdims=True)
        pv = lax.dot_general(e, v[j * C:(j + 1) * C, :],
                             (((1,), (0,)), ((), ())),
                             preferred_element_type=jnp.float32)
        acc = (pv + cd * acc) * (jnp.float32(1.0) / den_new)
        m_prev = m_new
        den_prev = den_new
    out_ref[...] = acc[None]


def _attention(xs, Wq, bq, Wk, bk, Wv, bv):
    bq3 = bq.reshape(N_HEADS, 1, D_HEAD)
    bk3 = bk.reshape(N_HEADS, 1, D_HEAD)
    bv3 = bv.reshape(N_HEADS, 1, D_HEAD)
    xspec = pl.BlockSpec((Q_BLK, D_MODEL), lambda tb, h: (tb, 0))
    wspec = pl.BlockSpec((D_HEAD, D_MODEL), lambda tb, h: (h, 0))
    bspec = pl.BlockSpec((1, 1, D_HEAD), lambda tb, h: (h, 0, 0))
    hsd = pl.BlockSpec((1, Q_BLK, D_HEAD), lambda tb, h: (h, tb, 0))
    q, k, v = pl.pallas_call(
        _qkv_body,
        grid=(S // Q_BLK, N_HEADS),
        in_specs=[xspec, wspec, wspec, wspec, bspec, bspec, bspec],
        out_specs=[hsd, hsd, hsd],
        out_shape=[jax.ShapeDtypeStruct((N_HEADS, S, D_HEAD), jnp.float32)] * 3,
        compiler_params=pltpu.CompilerParams(
            dimension_semantics=("arbitrary", "arbitrary")),
    )(xs, Wq, Wk, Wv, bq3, bk3, bv3)
    return pl.pallas_call(
        _attn_body,
        grid=(N_HEADS, S // Q_BLK),
        in_specs=[
            pl.BlockSpec((1, Q_BLK, D_HEAD), lambda h, qb: (h, qb, 0)),
            pl.BlockSpec((1, S, D_HEAD), lambda h, qb: (h, 0, 0)),
            pl.BlockSpec((1, S, D_HEAD), lambda h, qb: (h, 0, 0)),
        ],
        out_specs=pl.BlockSpec((1, Q_BLK, D_HEAD), lambda h, qb: (h, qb, 0)),
        out_shape=jax.ShapeDtypeStruct((N_HEADS, S, D_HEAD), jnp.float32),
        compiler_params=pltpu.CompilerParams(
            dimension_semantics=("arbitrary", "arbitrary")),
    )(q, k, v)


def _nf_body(a_ref, b_ref, o_ref):
    o_ref[...] = lax.dot_general(a_ref[...], b_ref[...],
                                 (((1,), (0,)), ((), ())),
                                 preferred_element_type=jnp.float32)


def _compose_nf(neuron_A, neuron_B):
    return pl.pallas_call(
        _nf_body,
        grid=(8,),
        in_specs=[
            pl.BlockSpec((N_NEURONS // 8, RANK), lambda i: (i, 0)),
            pl.BlockSpec((RANK, D_MODEL), lambda i: (0, 0)),
        ],
        out_specs=pl.BlockSpec((N_NEURONS // 8, D_MODEL), lambda i: (i, 0)),
        out_shape=jax.ShapeDtypeStruct((N_NEURONS, D_MODEL), jnp.float32),
        compiler_params=pltpu.CompilerParams(
            dimension_semantics=("arbitrary",)),
    )(neuron_A, neuron_B)


def _router_body(x_ref, c_ref, wp_ref, bp_ref, nf_ref, ti_ref, tw_ref,
                 sc_ref):
    nb = pl.program_id(1)
    x = x_ref[...]
    c = c_ref[...]
    comb = jnp.concatenate([x, c], axis=1)
    logit = (lax.dot_general(comb, wp_ref[...], _DN,
                             preferred_element_type=jnp.float32)
             + bp_ref[...])
    lm = jnp.max(logit, axis=1, keepdims=True)
    le = jnp.exp(logit - lm)
    w = le / jnp.sum(le, axis=1, keepdims=True)
    nf = nf_ref[...]
    ts = lax.dot_general(x, nf, _DN, preferred_element_type=jnp.float32)
    cs = lax.dot_general(c, nf, _DN, preferred_element_type=jnp.float32)
    sc_ref[nb] = w[:, 0:1] * ts + w[:, 1:2] * cs

    @pl.when(nb == N_NB - 1)
    def _():
        s = sc_ref[...]
        bidx = lax.broadcasted_iota(jnp.int32, (N_NB, TOK_BLK, NEU_BLK), 0)
        lidx = lax.broadcasted_iota(jnp.int32, (N_NB, TOK_BLK, NEU_BLK), 2)
        iota = bidx * NEU_BLK + lidx
        big = jnp.int32(N_NEURONS)
        vals, idxs = [], []
        for _i in range(K):
            m = jnp.max(jnp.max(s, axis=2), axis=0)
            mb = m[None, :, None]
            cidx = jnp.min(jnp.min(jnp.where(s == mb, iota, big), axis=2),
                           axis=0)
            cb = cidx[None, :, None]
            vals.append(m[:, None])
            idxs.append(cidx[:, None])
            s = jnp.where(iota == cb, NEG, s)
        tv = jnp.concatenate(vals, axis=1)
        ti = jnp.concatenate(idxs, axis=1)
        e = jnp.exp(tv - tv[:, 0:1])
        tw_ref[...] = e / jnp.sum(e, axis=1, keepdims=True)
        ti_ref[...] = ti


def _router(xs, ctx, Wp, bp2, nf):
    tok = pl.BlockSpec((TOK_BLK, D_MODEL), lambda t, n: (t, 0))
    return pl.pallas_call(
        _router_body,
        grid=(S // TOK_BLK, N_NB),
        in_specs=[
            tok, tok,
            pl.BlockSpec((2, 2 * D_MODEL), lambda t, n: (0, 0)),
            pl.BlockSpec((1, 2), lambda t, n: (0, 0)),
            pl.BlockSpec((NEU_BLK, D_MODEL), lambda t, n: (n, 0)),
        ],
        out_specs=[
            pl.BlockSpec((TOK_BLK, K), lambda t, n: (t, 0)),
            pl.BlockSpec((TOK_BLK, K), lambda t, n: (t, 0)),
        ],
        out_shape=[
            jax.ShapeDtypeStruct((S, K), jnp.int32),
            jax.ShapeDtypeStruct((S, K), jnp.float32),
        ],
        scratch_shapes=[pltpu.VMEM((N_NB, TOK_BLK, NEU_BLK), jnp.float32)],
        compiler_params=pltpu.CompilerParams(
            dimension_semantics=("arbitrary", "arbitrary")),
    )(xs, ctx, Wp, bp2, nf)


_NW = 32
_ROWS_PER_W = (S * K) // _NW
_CHUNK = 128
_GROW = 128


def _sc_gather_body(table_hbm, idx_hbm, out_hbm, idx_v, rows_v, sem):
    cid = lax.axis_index("c")
    sid = lax.axis_index("s")
    wid = sid * 2 + cid
    base = wid * _ROWS_PER_W
    for j in range(_ROWS_PER_W // _CHUNK):
        off = base + j * _CHUNK
        pltpu.sync_copy(idx_hbm.at[pl.ds(off, _CHUNK)], idx_v)
        pltpu.async_copy(table_hbm.at[idx_v], rows_v, sem).wait()
        pltpu.sync_copy(rows_v, out_hbm.at[pl.ds(off, _CHUNK)])


def _sc_gather(table_pad, idx_flat):
    mesh = plsc.VectorSubcoreMesh(core_axis_name="c", subcore_axis_name="s")
    fn = pl.kernel(
        _sc_gather_body,
        out_type=jax.ShapeDtypeStruct((S * K, _GROW), jnp.float32),
        mesh=mesh,
        scratch_types=[
            pltpu.VMEM((_CHUNK,), jnp.int32),
            pltpu.VMEM((_CHUNK, _GROW), jnp.float32),
            pltpu.SemaphoreType.DMA,
        ],
    )
    return fn(table_pad, idx_flat)


def _expand_body(g_ref, b_ref, out_ref):
    out_ref[...] = lax.dot_general(g_ref[...], b_ref[...],
                                   (((1,), (0,)), ((), ())),
                                   preferred_element_type=jnp.float32)


def _expand(gathered, neuron_B_pad):
    nblk = (S * K) // EXP_BLK
    return pl.pallas_call(
        _expand_body,
        grid=(nblk,),
        in_specs=[
            pl.BlockSpec((EXP_BLK, _GROW), lambda g: (g, 0)),
            pl.BlockSpec((_GROW, D_MODEL), lambda g: (0, 0)),
        ],
        out_specs=pl.BlockSpec((EXP_BLK, D_MODEL), lambda g: (g, 0)),
        out_shape=jax.ShapeDtypeStruct((S * K, D_MODEL), jnp.float32),
        compiler_params=pltpu.CompilerParams(
            dimension_semantics=("arbitrary",)),
    )(gathered, neuron_B_pad)


def kernel(x, neuron_A, neuron_B, Wq, bq, Wk, bk, Wv, bv, Wp, bp):
    xs = x[0]
    ctx_h = _attention(xs, Wq, bq, Wk, bk, Wv, bv)
    ctx = ctx_h.transpose(1, 0, 2).reshape(S, D_MODEL)
    nf = _compose_nf(neuron_A, neuron_B)
    ti, tw = _router(xs, ctx, Wp, bp.reshape(1, 2), nf)
    idx_flat = ti.reshape(S * K)
    A_pad = jnp.pad(neuron_A, ((0, 0), (0, _GROW - RANK)))
    B_pad = jnp.pad(neuron_B, ((0, _GROW - RANK), (0, 0)))
    gathered = _sc_gather(A_pad, idx_flat)
    selected = _expand(gathered, B_pad)
    return (selected.reshape(1, S, K, D_MODEL),
            ti.reshape(1, S, K),
            tw.reshape(1, S, K),
            ctx.reshape(1, S, D_MODEL))
